# baseline (speedup 1.0000x reference)
import functools

import jax
import jax.numpy as jnp
from jax.experimental import pallas as pl
from jax.experimental.pallas import tpu as pltpu

B, L, N, D = 32, 96, 321, 32
E, K, DFF = 4, 2, 64
PATCHES = [8, 6, 4, 2]


def _ntn(x, nan=0.0, posinf=0.0, neginf=0.0):
    return jnp.nan_to_num(x, nan=nan, posinf=posinf, neginf=neginf)


def _moving_avg(x, k):
    front = jnp.repeat(x[:, :1, :], (k - 1) // 2, axis=1)
    end = jnp.repeat(x[:, -1:, :], k // 2, axis=1)
    xp = jnp.concatenate([front, x, end], axis=1)
    c = jnp.cumsum(jnp.pad(xp, ((0, 0), (1, 0), (0, 0))), axis=1)
    return (c[:, k:, :] - c[:, :-k, :]) / k


def _trend(x):
    means = [_moving_avg(x, k) for k in (4, 8, 12)]
    return sum(means) / len(means)


def _fourier_season(x, topk=3, low_freq=1):
    b, t, d = x.shape
    xf = jnp.fft.rfft(x, axis=1)
    if t % 2 == 0:
        xf = xf[:, low_freq:-1]
        f = jnp.fft.rfftfreq(t)[low_freq:-1]
    else:
        xf = xf[:, low_freq:]
        f = jnp.fft.rfftfreq(t)[low_freq:]
    mag = jnp.abs(xf)
    _, idx = jax.lax.top_k(jnp.moveaxis(mag, 1, 2), topk)
    idx = jnp.moveaxis(idx, 2, 1)
    xf_top = jnp.take_along_axis(xf, idx, axis=1)
    f_top = f[idx]
    xf_full = jnp.concatenate([xf_top, jnp.conj(xf_top)], axis=1)
    f_full = jnp.concatenate([f_top, -f_top], axis=1)
    tv = jnp.arange(t, dtype=jnp.float32)[None, None, :, None]
    amp = (jnp.abs(xf_full) / t)[:, :, None, :]
    phase = jnp.angle(xf_full)[:, :, None, :]
    xt = amp * jnp.cos(2.0 * jnp.pi * f_full[:, :, None, :] * tv + phase)
    return xt.sum(axis=1)


def _cv_squared(v):
    return jnp.var(v, ddof=1) / (jnp.mean(v) ** 2 + 1e-10)


def _prob_in_top_k(clean, noisy, std, top_vals, k):
    b = clean.shape[0]
    m = top_vals.shape[1]
    flat = top_vals.reshape(-1)
    pos_in = jnp.clip(jnp.arange(b) * m + k, 0, flat.shape[0] - 1)
    thr_in = flat[pos_in][:, None]
    is_in = noisy > thr_in
    pos_out = jnp.clip(pos_in - 1, 0, flat.shape[0] - 1)
    thr_out = flat[pos_out][:, None]
    sstd = jnp.clip(_ntn(jnp.clip(std, 1e-06), 1e-06, 1000000.0, 1e-06), 1e-06)
    ni = jnp.clip(_ntn((clean - thr_in) / sstd, 0.0, 10.0, -10.0), -10.0, 10.0)
    no = jnp.clip(_ntn((clean - thr_out) / sstd, 0.0, 10.0, -10.0), -10.0, 10.0)
    pi = jax.scipy.stats.norm.cdf(ni)
    po = jax.scipy.stats.norm.cdf(no)
    return jnp.where(is_in, pi, po)


def _expert(x, p, W1, b1, W2, b2, Wq, Wk, Wv, Wo):
    b, t, n, d = x.shape
    pn = t // p
    h = jax.nn.gelu(x @ W1 + b1) @ W2 + b2
    x1 = x + h
    xp = x1.reshape(b, pn, p, n, d).mean(axis=2)
    q = xp @ Wq
    kk = xp @ Wk
    v = xp @ Wv
    attn = jax.nn.softmax(jnp.einsum('bpnd,bqnd->bnpq', q, kk) / jnp.sqrt(float(d)), axis=-1)
    o = jnp.einsum('bnpq,bqnd->bpnd', attn, v) @ Wo
    return x1 + jnp.repeat(o, p, axis=1)


def _combine_kernel(x_ref, eo_ref, g_ref, out_ref):
    b = pl.program_id(0)
    e = pl.program_id(1)
    contrib = g_ref[b, e] * eo_ref[0]

    @pl.when(e == 0)
    def _init():
        out_ref[...] = x_ref[...] + contrib

    @pl.when(e > 0)
    def _acc():
        out_ref[...] = out_ref[...] + contrib


def kernel(x, W_start, b_start, W_gate, b_gate, W_noise, b_noise, W1, b1, W2, b2, Wq, Wk, Wv, Wo):
    x3 = x[:, :, :, 0]
    trend = _trend(x3)
    season = _fourier_season(x3)
    new_x = _ntn(x3 + season + trend)
    xs = _ntn(jnp.einsum('bln,on->blo', new_x, W_start)[:, :, 0] + b_start)
    clean = _ntn(xs @ W_gate.T + b_gate)
    raw_noise = xs @ W_noise.T + b_noise
    std = jnp.clip(_ntn(jax.nn.softplus(raw_noise) + 0.01, 0.01, 1000000.0, 0.01), 0.01)
    noise = jax.random.normal(jax.random.key(42), clean.shape, dtype=jnp.float32)
    noisy = _ntn(clean + noise * std)
    logits = noisy
    m = min(K + 1, E)
    top_vals, top_idx = jax.lax.top_k(logits, m)
    tkl = top_vals[:, :K]
    tki = top_idx[:, :K]
    gk = _ntn(jax.nn.softmax(tkl, axis=1))
    gates = jnp.zeros_like(logits).at[jnp.arange(logits.shape[0])[:, None], tki].set(gk)
    gates = _ntn(gates)
    load = _prob_in_top_k(clean, noisy, std, top_vals, K).sum(axis=0)
    importance = gates.sum(axis=0)
    balance_loss = (_cv_squared(importance) + _cv_squared(load)) * 0.01

    eo_list = []
    for e in range(E):
        eo = _expert(x, PATCHES[e], W1[e], b1[e], W2[e], b2[e], Wq[e], Wk[e], Wv[e], Wo[e])
        eo_list.append(eo)
    eo_stack = jnp.stack(eo_list, axis=0)
    F = L * N * D // 128
    x_flat = x.reshape(B, F, 128)
    eo_flat = eo_stack.reshape(E, B, F, 128)

    out = pl.pallas_call(
        _combine_kernel,
        grid=(B, E),
        in_specs=[
            pl.BlockSpec((1, F, 128), lambda b, e: (b, 0, 0)),
            pl.BlockSpec((1, 1, F, 128), lambda b, e: (e, b, 0, 0)),
            pl.BlockSpec(memory_space=pltpu.SMEM),
        ],
        out_specs=pl.BlockSpec((1, F, 128), lambda b, e: (b, 0, 0)),
        out_shape=jax.ShapeDtypeStruct((B, F, 128), jnp.float32),
    )(x_flat, eo_flat, gates)
    return (out.reshape(B, L, N, D), balance_loss)


# speedup vs baseline: 1.4570x; 1.4570x over previous
import functools
import math

import jax
import jax.numpy as jnp
from jax.experimental import pallas as pl
from jax.experimental.pallas import tpu as pltpu

B, L, N, D = 32, 96, 321, 32
E, K, DFF = 4, 2, 64
PATCHES = (8, 6, 4, 2)
NP = 328
PN = 48
G = 8
NG = NP // G
T = L * NP
JCOLS = NP * PN


def _ntn(x, nan=0.0, posinf=0.0, neginf=0.0):
    return jnp.nan_to_num(x, nan=nan, posinf=posinf, neginf=neginf)


def _moving_avg(x, k):
    front = jnp.repeat(x[:, :1, :], (k - 1) // 2, axis=1)
    end = jnp.repeat(x[:, -1:, :], k // 2, axis=1)
    xp = jnp.concatenate([front, x, end], axis=1)
    c = jnp.cumsum(jnp.pad(xp, ((0, 0), (1, 0), (0, 0))), axis=1)
    return (c[:, k:, :] - c[:, :-k, :]) / k


def _trend(x):
    means = [_moving_avg(x, k) for k in (4, 8, 12)]
    return sum(means) / len(means)


def _fourier_season(x, topk=3, low_freq=1):
    b, t, d = x.shape
    xf = jnp.fft.rfft(x, axis=1)
    xf = xf[:, low_freq:-1]
    f = jnp.fft.rfftfreq(t)[low_freq:-1]
    mag = jnp.abs(xf)
    _, idx = jax.lax.top_k(jnp.moveaxis(mag, 1, 2), topk)
    idx = jnp.moveaxis(idx, 2, 1)
    xf_top = jnp.take_along_axis(xf, idx, axis=1)
    f_top = f[idx]
    xf_full = jnp.concatenate([xf_top, jnp.conj(xf_top)], axis=1)
    f_full = jnp.concatenate([f_top, -f_top], axis=1)
    tv = jnp.arange(t, dtype=jnp.float32)[None, None, :, None]
    amp = (jnp.abs(xf_full) / t)[:, :, None, :]
    phase = jnp.angle(xf_full)[:, :, None, :]
    xt = amp * jnp.cos(2.0 * jnp.pi * f_full[:, :, None, :] * tv + phase)
    return xt.sum(axis=1)


def _cv_squared(v):
    return jnp.var(v, ddof=1) / (jnp.mean(v) ** 2 + 1e-10)


def _prob_in_top_k(clean, noisy, std, top_vals, k):
    b = clean.shape[0]
    m = top_vals.shape[1]
    flat = top_vals.reshape(-1)
    pos_in = jnp.clip(jnp.arange(b) * m + k, 0, flat.shape[0] - 1)
    thr_in = flat[pos_in][:, None]
    is_in = noisy > thr_in
    pos_out = jnp.clip(pos_in - 1, 0, flat.shape[0] - 1)
    thr_out = flat[pos_out][:, None]
    sstd = jnp.clip(_ntn(jnp.clip(std, 1e-06), 1e-06, 1000000.0, 1e-06), 1e-06)
    ni = jnp.clip(_ntn((clean - thr_in) / sstd, 0.0, 10.0, -10.0), -10.0, 10.0)
    no = jnp.clip(_ntn((clean - thr_out) / sstd, 0.0, 10.0, -10.0), -10.0, 10.0)
    pi = jax.scipy.stats.norm.cdf(ni)
    po = jax.scipy.stats.norm.cdf(no)
    return jnp.where(is_in, pi, po)


def _dot00(a, b):
    return jax.lax.dot_general(a, b, (((0,), (0,)), ((), ())),
                               preferred_element_type=jnp.float32)


def _dot11(a, b):
    return jax.lax.dot_general(a, b, (((1,), (1,)), ((), ())),
                               preferred_element_type=jnp.float32)


def _cond4(e, f0, f1, f2, f3):
    return jax.lax.cond(
        e == 0, f0,
        lambda: jax.lax.cond(e == 1, f1,
                             lambda: jax.lax.cond(e == 2, f2, f3)))


def _expert_kernel(tki_ref, gk_ref, x_ref, w1_ref, b1_ref, w2_ref, b2_ref,
                   wq_ref, wk_ref, wv_ref, wo_ref, out_ref, o_scr):
    bb = pl.program_id(0)
    kk_ = pl.program_id(1)
    e = tki_ref[bb, kk_]
    g = gk_ref[bb, kk_]
    pn_e = jnp.where(e == 0, L // PATCHES[0],
                     jnp.where(e == 1, L // PATCHES[1],
                               jnp.where(e == 2, L // PATCHES[2],
                                         L // PATCHES[3])))

    xT = x_ref[0]
    b1c = jnp.transpose(b1_ref[0], (1, 0))
    b2c = jnp.transpose(b2_ref[0], (1, 0))
    h1 = jax.nn.gelu(_dot00(w1_ref[0], xT) + b1c)
    h2 = _dot00(w2_ref[0], h1) + b2c
    x1 = xT + h2

    def pool(p):
        pn = L // p
        v = x1.reshape(D, pn, p, NP).sum(axis=2) * (1.0 / p)
        if pn < PN:
            v = jnp.concatenate(
                [v, jnp.zeros((D, PN - pn, NP), jnp.float32)], axis=1)
        return v

    xp = _cond4(e, lambda: pool(PATCHES[0]), lambda: pool(PATCHES[1]),
                lambda: pool(PATCHES[2]), lambda: pool(PATCHES[3]))

    xpn = jnp.transpose(xp, (0, 2, 1)).reshape(D, JCOLS)
    q = _dot00(wq_ref[0], xpn)
    kq = _dot00(wk_ref[0], xpn)
    v = _dot00(wv_ref[0], xpn)

    scale = 1.0 / math.sqrt(float(D))
    rows_n = jax.lax.broadcasted_iota(jnp.int32, (G * PN, G * PN), 0) // PN
    cols_n = jax.lax.broadcasted_iota(jnp.int32, (G * PN, G * PN), 1) // PN
    cols_j = jax.lax.broadcasted_iota(jnp.int32, (G * PN, G * PN), 1) % PN
    base_mask = rows_n == cols_n
    mask = base_mask & (cols_j < pn_e)

    for gi in range(NG):
        lo = gi * G * PN
        hi = lo + G * PN
        qg = q[:, lo:hi]
        kg = kq[:, lo:hi]
        vg = v[:, lo:hi]
        sc = _dot00(qg, kg) * scale
        sc = jnp.where(mask, sc, -1e30)
        sc = sc - jnp.max(sc, axis=1, keepdims=True)
        ex = jnp.exp(sc)
        at = ex / jnp.sum(ex, axis=1, keepdims=True)
        o_scr[:, lo:hi] = _dot11(vg, at)

    o2 = _dot00(wo_ref[0], o_scr[...])
    oj = jnp.transpose(o2.reshape(D, NP, PN), (0, 2, 1))

    def rep(p):
        pn = L // p
        r = jnp.broadcast_to(oj[:, :pn, None, :], (D, pn, p, NP))
        return r.reshape(D, T)

    ro = _cond4(e, lambda: rep(PATCHES[0]), lambda: rep(PATCHES[1]),
                lambda: rep(PATCHES[2]), lambda: rep(PATCHES[3]))

    contrib = g * (x1 + ro)

    @pl.when(kk_ == 0)
    def _init():
        out_ref[0] = xT + contrib

    @pl.when(kk_ != 0)
    def _acc():
        out_ref[0] = out_ref[0] + contrib


def _run_experts(x_t, tki, gk, W1, b1, W2, b2, Wq, Wk, Wv, Wo):
    grid_spec = pltpu.PrefetchScalarGridSpec(
        num_scalar_prefetch=1,
        grid=(B, K),
        in_specs=[
            pl.BlockSpec(memory_space=pltpu.SMEM),
            pl.BlockSpec((1, D, T), lambda b, k, tki: (b, 0, 0)),
            pl.BlockSpec((1, D, DFF), lambda b, k, tki: (tki[b, k], 0, 0)),
            pl.BlockSpec((1, 1, DFF), lambda b, k, tki: (tki[b, k], 0, 0)),
            pl.BlockSpec((1, DFF, D), lambda b, k, tki: (tki[b, k], 0, 0)),
            pl.BlockSpec((1, 1, D), lambda b, k, tki: (tki[b, k], 0, 0)),
            pl.BlockSpec((1, D, D), lambda b, k, tki: (tki[b, k], 0, 0)),
            pl.BlockSpec((1, D, D), lambda b, k, tki: (tki[b, k], 0, 0)),
            pl.BlockSpec((1, D, D), lambda b, k, tki: (tki[b, k], 0, 0)),
            pl.BlockSpec((1, D, D), lambda b, k, tki: (tki[b, k], 0, 0)),
        ],
        out_specs=pl.BlockSpec((1, D, T), lambda b, k, tki: (b, 0, 0)),
        scratch_shapes=[pltpu.VMEM((D, JCOLS), jnp.float32)],
    )
    return pl.pallas_call(
        _expert_kernel,
        grid_spec=grid_spec,
        out_shape=jax.ShapeDtypeStruct((B, D, T), jnp.float32),
        compiler_params=pltpu.CompilerParams(
            dimension_semantics=("arbitrary", "arbitrary")),
    )(tki, gk, x_t, W1, b1, W2, b2, Wq, Wk, Wv, Wo)


def kernel(x, W_start, b_start, W_gate, b_gate, W_noise, b_noise, W1, b1, W2, b2, Wq, Wk, Wv, Wo):
    x3 = x[:, :, :, 0]
    trend = _trend(x3)
    season = _fourier_season(x3)
    new_x = _ntn(x3 + season + trend)
    xs = _ntn(jnp.einsum('bln,on->blo', new_x, W_start)[:, :, 0] + b_start)
    clean = _ntn(xs @ W_gate.T + b_gate)
    raw_noise = xs @ W_noise.T + b_noise
    std = jnp.clip(_ntn(jax.nn.softplus(raw_noise) + 0.01, 0.01, 1000000.0, 0.01), 0.01)
    noise = jax.random.normal(jax.random.key(42), clean.shape, dtype=jnp.float32)
    noisy = _ntn(clean + noise * std)
    m = min(K + 1, E)
    top_vals, top_idx = jax.lax.top_k(noisy, m)
    tkl = top_vals[:, :K]
    tki = top_idx[:, :K].astype(jnp.int32)
    gk = _ntn(jax.nn.softmax(tkl, axis=1))
    gates = jnp.zeros_like(noisy).at[jnp.arange(B)[:, None], tki].set(gk)
    gates = _ntn(gates)
    load = _prob_in_top_k(clean, noisy, std, top_vals, K).sum(axis=0)
    importance = gates.sum(axis=0)
    balance_loss = (_cv_squared(importance) + _cv_squared(load)) * 0.01

    x_t = jnp.pad(jnp.transpose(x, (0, 3, 1, 2)),
                  ((0, 0), (0, 0), (0, 0), (0, NP - N))).reshape(B, D, T)
    b1r = b1.reshape(E, 1, DFF)
    b2r = b2.reshape(E, 1, D)
    out_t = _run_experts(x_t, tki, gk, W1, b1r, W2, b2r, Wq, Wk, Wv, Wo)
    out = jnp.transpose(out_t.reshape(B, D, L, NP), (0, 2, 3, 1))[:, :, :N, :]
    return (out, balance_loss)
